# Initial kernel scaffold; baseline (speedup 1.0000x reference)
import jax
import jax.numpy as jnp
from jax.experimental import pallas as pl


def kernel(x, pos, edge_index, W_in, W_out, W_lin, b_lin, W_src, b_src, W_dst, b_dst, Wp1, bp1, Wp2, bp2, Wa1, ba1, Wa2, ba2):
    raise NotImplementedError("write your pallas kernel here")


# speedup vs baseline: 4.8020x; 4.8020x over previous
import functools

import jax
import jax.numpy as jnp
from jax import lax
from jax.experimental import pallas as pl
from jax.experimental.pallas import tpu as pltpu
from jax.experimental.pallas import tpu_sc as plsc

N = 10000
D = 128
E0 = 320000
NP = 10240
NC = 2
NS = 16
NW = NC * NS
PTE = 10368
EP = PTE * NW
CH = 128
NCHUNK = PTE // CH
RPT = NP // NS
EBLK = 4096
EGRID = EP // EBLK
NDUMP = NP - N
EPS = 1e-5

_mesh = plsc.VectorSubcoreMesh(core_axis_name="c", subcore_axis_name="s")


def _tc_pre(x_ref, wi, ws, bs, wd, bd, wl, bl, as_o, ad_o, hl_o):
  t = jnp.dot(x_ref[...], wi[...], preferred_element_type=jnp.float32)
  m = jnp.mean(t, axis=0, keepdims=True)
  c = t - m
  v = jnp.mean(c * c, axis=0, keepdims=True)
  h = jnp.maximum(c * lax.rsqrt(v + EPS), 0.0)
  as_o[...] = jnp.dot(h, ws[...], preferred_element_type=jnp.float32) + bs[...]
  ad_o[...] = jnp.dot(h, wd[...], preferred_element_type=jnp.float32) + bd[...]
  hl_o[...] = jnp.dot(h, wl[...], preferred_element_type=jnp.float32) + bl[...]


def _tc_c0(al_ref, out_ref, acc):
  i = pl.program_id(0)

  @pl.when(i == 0)
  def _():
    acc[...] = jnp.zeros_like(acc)

  a = al_ref[...]
  acc[0:1, :] += jnp.sum(a, axis=0, keepdims=True)
  acc[1:2, :] += jnp.sum(a * a, axis=0, keepdims=True)

  @pl.when(i == EGRID - 1)
  def _():
    out_ref[...] = acc[...]


def _tc_c1(al_ref, st, wa1, ba1, a1_o, st_o, acc):
  i = pl.program_id(0)

  @pl.when(i == 0)
  def _():
    acc[...] = jnp.zeros_like(acc)

  a = al_ref[...]
  z = jnp.maximum((a - st[0:1, :]) * st[1:2, :], 0.0)
  a1 = jnp.dot(z, wa1[...], preferred_element_type=jnp.float32) + ba1[...]
  a1_o[...] = a1
  acc[0:1, :] += jnp.sum(a1, axis=0, keepdims=True)
  acc[1:2, :] += jnp.sum(a1 * a1, axis=0, keepdims=True)

  @pl.when(i == EGRID - 1)
  def _():
    st_o[...] = acc[...]


def _tc_c2(a1_ref, st, wa2, ba2, ae_o):
  a1 = a1_ref[...]
  z = jnp.maximum((a1 - st[0:1, :]) * st[1:2, :], 0.0)
  a2 = jnp.dot(z, wa2[...], preferred_element_type=jnp.float32) + ba2[...]
  ae_o[...] = jnp.exp(a2)


def _tc_fin(p0, p1, cnt, x_ref, wo, out_ref):
  s = p0[...] + p1[...]
  o = s / jnp.maximum(cnt[...], 1.0)
  m = jnp.mean(o, axis=0, keepdims=True)
  c = o - m
  v = jnp.mean(c * c, axis=0, keepdims=True)
  o = jnp.maximum(c * lax.rsqrt(v + EPS), 0.0)
  o = jnp.dot(o, wo[...], preferred_element_type=jnp.float32)
  m = jnp.mean(o, axis=0, keepdims=True)
  c = o - m
  v = jnp.mean(c * c, axis=0, keepdims=True)
  o = c * lax.rsqrt(v + EPS)
  out_ref[...] = jnp.maximum(o + x_ref[...], 0.0)


def _wid():
  return lax.axis_index("s") * NC + lax.axis_index("c")


def _pos_rel(px, py, pz, sv, dv):
  rx = plsc.load_gather(px, [dv]) - plsc.load_gather(px, [sv])
  ry = plsc.load_gather(py, [dv]) - plsc.load_gather(py, [sv])
  rz = plsc.load_gather(pz, [dv]) - plsc.load_gather(pz, [sv])
  return rx, ry, rz


def _sc_a(px_h, py_h, pz_h, src_h, dst_h, msk_h, prm_h, out_h,
          px, py, pz, sidx, didx, mskv, prm, obuf):
  base = _wid() * PTE
  pltpu.sync_copy(px_h, px)
  pltpu.sync_copy(py_h, py)
  pltpu.sync_copy(pz_h, pz)
  pltpu.sync_copy(prm_h, prm)
  va = prm[pl.ds(0, 16)]
  w = [va[i] for i in range(9)]
  bp = [va[9 + i] for i in range(3)]
  zero = jnp.zeros((16,), jnp.float32)

  def chunk(ci, carry):
    off = base + ci * CH
    pltpu.sync_copy(src_h.at[pl.ds(off, CH)], sidx)
    pltpu.sync_copy(dst_h.at[pl.ds(off, CH)], didx)
    pltpu.sync_copy(msk_h.at[pl.ds(off, CH)], mskv)

    def blk(b, c2):
      s0, s1, s2, q0, q1, q2, nl = c2
      sv = sidx[pl.ds(b * 16, 16)]
      dv = didx[pl.ds(b * 16, 16)]
      mv = mskv[pl.ds(b * 16, 16)]
      rx, ry, rz = _pos_rel(px, py, pz, sv, dv)
      t0 = rx * w[0] + ry * w[1] + rz * w[2] + bp[0]
      t1 = rx * w[3] + ry * w[4] + rz * w[5] + bp[1]
      t2 = rx * w[6] + ry * w[7] + rz * w[8] + bp[2]
      tm0 = t0 * mv
      tm1 = t1 * mv
      tm2 = t2 * mv
      return (s0 + tm0, s1 + tm1, s2 + tm2,
              q0 + tm0 * t0, q1 + tm1 * t1, q2 + tm2 * t2, nl + mv)

    return lax.fori_loop(0, CH // 16, blk, carry)

  acc = lax.fori_loop(0, NCHUNK, chunk, (zero,) * 7)
  for i in range(7):
    obuf[pl.ds(i * 16, 16)] = acc[i]
  pltpu.sync_copy(obuf, out_h.at[_wid()])


def _sc_b(asrc_h, adst_h, px_h, py_h, pz_h, src_h, dst_h, msk_h, prm_h, wp2_h,
          alpha_h, r0_h, r1_h, r2_h,
          px, py, pz, sidx, didx, mskv, prm, wp2, bufS, bufD, abuf,
          rb0, rb1, rb2, sem, sem2):
  base = _wid() * PTE
  pltpu.sync_copy(px_h, px)
  pltpu.sync_copy(py_h, py)
  pltpu.sync_copy(pz_h, pz)
  pltpu.sync_copy(prm_h, prm)
  pltpu.sync_copy(wp2_h, wp2)
  va = prm[pl.ds(0, 16)]
  vb = prm[pl.ds(16, 16)]
  w = [va[i] for i in range(9)]
  bp = [va[9 + i] for i in range(3)]
  m1 = [vb[i] for i in range(3)]
  iv1 = [vb[3 + i] for i in range(3)]
  w2 = [[wp2[j, pl.ds(p * 16, 16)] for p in range(8)] for j in range(4)]

  def chunk(ci, _):
    off = base + ci * CH
    pltpu.sync_copy(src_h.at[pl.ds(off, CH)], sidx)
    pltpu.sync_copy(dst_h.at[pl.ds(off, CH)], didx)
    pltpu.sync_copy(msk_h.at[pl.ds(off, CH)], mskv)
    cpS = pltpu.async_copy(asrc_h.at[sidx], bufS, sem)
    cpD = pltpu.async_copy(adst_h.at[didx], bufD, sem2)
    cpS.wait()
    cpD.wait()

    def blk(b, __):
      sv = sidx[pl.ds(b * 16, 16)]
      dv = didx[pl.ds(b * 16, 16)]
      mv = mskv[pl.ds(b * 16, 16)]
      rx, ry, rz = _pos_rel(px, py, pz, sv, dv)
      rr = []
      for j in range(3):
        t = rx * w[3 * j] + ry * w[3 * j + 1] + rz * w[3 * j + 2] + bp[j]
        rr.append(jnp.maximum((t - m1[j]) * iv1[j], 0.0))
      rb0[pl.ds(b * 16, 16)] = rr[0]
      rb1[pl.ds(b * 16, 16)] = rr[1]
      rb2[pl.ds(b * 16, 16)] = rr[2]
      for e in range(16):
        row = b * 16 + e
        r0 = rr[0][e]
        r1 = rr[1][e]
        r2 = rr[2][e]
        me = mv[e]
        for p in range(8):
          dp = r0 * w2[0][p] + r1 * w2[1][p] + r2 * w2[2][p] + w2[3][p]
          al = (bufD[row, pl.ds(p * 16, 16)]
                - bufS[row, pl.ds(p * 16, 16)] + dp) * me
          abuf[row, pl.ds(p * 16, 16)] = al
      return 0

    lax.fori_loop(0, CH // 16, blk, 0)
    pltpu.sync_copy(abuf, alpha_h.at[pl.ds(off, CH)])
    pltpu.sync_copy(rb0, r0_h.at[pl.ds(off, CH)])
    pltpu.sync_copy(rb1, r1_h.at[pl.ds(off, CH)])
    pltpu.sync_copy(rb2, r2_h.at[pl.ds(off, CH)])
    return 0

  lax.fori_loop(0, NCHUNK, chunk, 0)


def _sc_d(ae_h, dst_h, msk_h, asum_o, cnt_o,
          asum_sh, cnt_sh, aev, didx, mskv, zb, zb1):
  cid = lax.axis_index("c")
  sid = lax.axis_index("s")
  base = _wid() * PTE
  zv = jnp.zeros((16,), jnp.float32)

  def zrow(i, _):
    zb[i, :] = zv
    return 0

  lax.fori_loop(0, RPT, zrow, 0)

  def zrow1(i, _):
    zb1[pl.ds(i * 16, 16)] = zv
    return 0

  lax.fori_loop(0, RPT // 16, zrow1, 0)
  pltpu.sync_copy(zb, asum_sh.at[pl.ds(sid * RPT, RPT)])
  pltpu.sync_copy(zb1, cnt_sh.at[pl.ds(sid * RPT, RPT)])
  plsc.subcore_barrier()

  def chunk(ci, _):
    off = base + ci * CH
    pltpu.sync_copy(ae_h.at[pl.ds(off, CH)], aev)
    pltpu.sync_copy(dst_h.at[pl.ds(off, CH)], didx)
    pltpu.sync_copy(msk_h.at[pl.ds(off, CH)], mskv)
    pltpu.sync_copy(aev, asum_sh.at[didx], add=True)
    pltpu.sync_copy(mskv, cnt_sh.at[didx], add=True)
    return 0

  lax.fori_loop(0, NCHUNK, chunk, 0)
  plsc.subcore_barrier()
  pltpu.sync_copy(asum_sh.at[pl.ds(sid * RPT, RPT)], zb)
  pltpu.sync_copy(zb, asum_o.at[cid, pl.ds(sid * RPT, RPT)])
  pltpu.sync_copy(cnt_sh.at[pl.ds(sid * RPT, RPT)], zb1)
  pltpu.sync_copy(zb1, cnt_o.at[cid, pl.ds(sid * RPT, RPT)])


def _sc_e(hlin_h, ae_h, asum_h, r0_h, r1_h, r2_h, src_h, dst_h, wp2_h,
          out_o,
          out_sh, sidx, didx, wp2, aev, asv, bufH, msgb,
          rb0, rb1, rb2, sem, sem2):
  cid = lax.axis_index("c")
  sid = lax.axis_index("s")
  base = _wid() * PTE
  pltpu.sync_copy(wp2_h, wp2)
  w2 = [[wp2[j, pl.ds(p * 16, 16)] for p in range(8)] for j in range(4)]
  zv = jnp.zeros((16,), jnp.float32)

  def zrow(i, _):
    for p in range(8):
      msgb[i, pl.ds(p * 16, 16)] = zv
    return 0

  lax.fori_loop(0, CH, zrow, 0)
  for r in range(RPT // CH):
    pltpu.sync_copy(msgb, out_sh.at[pl.ds(sid * RPT + r * CH, CH)])
  plsc.subcore_barrier()

  def chunk(ci, _):
    off = base + ci * CH
    pltpu.sync_copy(src_h.at[pl.ds(off, CH)], sidx)
    pltpu.sync_copy(dst_h.at[pl.ds(off, CH)], didx)
    pltpu.sync_copy(ae_h.at[pl.ds(off, CH)], aev)
    pltpu.sync_copy(r0_h.at[pl.ds(off, CH)], rb0)
    pltpu.sync_copy(r1_h.at[pl.ds(off, CH)], rb1)
    pltpu.sync_copy(r2_h.at[pl.ds(off, CH)], rb2)
    cpH = pltpu.async_copy(hlin_h.at[sidx], bufH, sem)
    cpA = pltpu.async_copy(asum_h.at[didx], asv, sem2)
    cpH.wait()
    cpA.wait()

    def blk(b, __):
      rr = [rb0[pl.ds(b * 16, 16)], rb1[pl.ds(b * 16, 16)],
            rb2[pl.ds(b * 16, 16)]]
      for e in range(16):
        row = b * 16 + e
        r0 = rr[0][e]
        r1 = rr[1][e]
        r2 = rr[2][e]
        att = aev[row, :] / (asv[row, :] + 1e-16)
        for p in range(8):
          dp = r0 * w2[0][p] + r1 * w2[1][p] + r2 * w2[2][p] + w2[3][p]
          msgb[row, pl.ds(p * 16, 16)] = att * (
              bufH[row, pl.ds(p * 16, 16)] + dp)
      return 0

    lax.fori_loop(0, CH // 16, blk, 0)
    pltpu.sync_copy(msgb, out_sh.at[didx], add=True)
    return 0

  lax.fori_loop(0, NCHUNK, chunk, 0)
  plsc.subcore_barrier()
  for r in range(RPT // CH):
    pltpu.sync_copy(out_sh.at[pl.ds(sid * RPT + r * CH, CH)], msgb)
    pltpu.sync_copy(msgb, out_o.at[cid, pl.ds(sid * RPT + r * CH, CH)])


def kernel(x, pos, edge_index, W_in, W_out, W_lin, b_lin, W_src, b_src,
           W_dst, b_dst, Wp1, bp1, Wp2, bp2, Wa1, ba1, Wa2, ba2):
  f32 = jnp.float32

  src0, dst0 = edge_index[0], edge_index[1]
  keep = src0 != dst0
  loops = jnp.arange(N, dtype=jnp.int32)
  npad = EP - (E0 + N)
  dump0 = N + (jnp.arange(E0, dtype=jnp.int32) % NDUMP)
  dumpP = N + (jnp.arange(npad, dtype=jnp.int32) % NDUMP)
  src = jnp.concatenate([src0, loops, jnp.zeros((npad,), jnp.int32)])
  dst = jnp.concatenate([jnp.where(keep, dst0, dump0), loops, dumpP])
  msk = jnp.concatenate([keep.astype(f32), jnp.ones((N,), f32),
                         jnp.zeros((npad,), f32)])

  sds = jax.ShapeDtypeStruct
  a_src, a_dst, h_lin = pl.pallas_call(
      _tc_pre,
      out_shape=[sds((N, D), f32)] * 3,
  )(x, W_in.T, W_src.T, b_src[None, :], W_dst.T, b_dst[None, :],
    W_lin.T, b_lin[None, :])

  pad_n = lambda a: jnp.pad(a, ((0, NP - N), (0, 0)))
  a_src_p = pad_n(a_src)
  a_dst_p = pad_n(a_dst)
  hlin_p = pad_n(h_lin)
  posx = jnp.pad(pos[:, 0], (0, NP - N))
  posy = jnp.pad(pos[:, 1], (0, NP - N))
  posz = jnp.pad(pos[:, 2], (0, NP - N))

  prmA = jnp.concatenate([Wp1.reshape(-1), bp1, jnp.zeros((20,), f32)])
  partA = pl.kernel(
      _sc_a,
      out_type=sds((NW, 112), f32),
      mesh=_mesh,
      compiler_params=pltpu.CompilerParams(needs_layout_passes=False, use_tc_tiling_on_sc=False),
      scratch_types=[
          pltpu.VMEM((NP,), f32), pltpu.VMEM((NP,), f32),
          pltpu.VMEM((NP,), f32),
          pltpu.VMEM((CH,), jnp.int32), pltpu.VMEM((CH,), jnp.int32),
          pltpu.VMEM((CH,), f32),
          pltpu.VMEM((32,), f32), pltpu.VMEM((112,), f32),
      ],
  )(posx, posy, posz, src, dst, msk, prmA)

  tot = jnp.sum(partA.reshape(NW, 7, 16), axis=(0, 2))
  n_live = tot[6]
  m1 = tot[0:3] / n_live
  v1 = tot[3:6] / n_live - m1 * m1
  iv1 = lax.rsqrt(v1 + EPS)

  prmB = jnp.concatenate([Wp1.reshape(-1), bp1, jnp.zeros((4,), f32),
                          m1, iv1, jnp.zeros((10,), f32)])
  wp2t = jnp.concatenate([Wp2.T, bp2[None, :]])
  alpha, r0a, r1a, r2a = pl.kernel(
      _sc_b,
      out_type=[sds((EP, D), f32), sds((EP,), f32), sds((EP,), f32),
                sds((EP,), f32)],
      mesh=_mesh,
      compiler_params=pltpu.CompilerParams(needs_layout_passes=False, use_tc_tiling_on_sc=False),
      scratch_types=[
          pltpu.VMEM((NP,), f32), pltpu.VMEM((NP,), f32),
          pltpu.VMEM((NP,), f32),
          pltpu.VMEM((CH,), jnp.int32), pltpu.VMEM((CH,), jnp.int32),
          pltpu.VMEM((CH,), f32),
          pltpu.VMEM((32,), f32), pltpu.VMEM((4, D), f32),
          pltpu.VMEM((CH, D), f32), pltpu.VMEM((CH, D), f32),
          pltpu.VMEM((CH, D), f32),
          pltpu.VMEM((CH,), f32), pltpu.VMEM((CH,), f32),
          pltpu.VMEM((CH,), f32),
          pltpu.SemaphoreType.DMA, pltpu.SemaphoreType.DMA,
      ],
  )(a_src_p, a_dst_p, posx, posy, posz, src, dst, msk, prmB, wp2t)

  stats2 = pl.pallas_call(
      _tc_c0,
      grid=(EGRID,),
      in_specs=[pl.BlockSpec((EBLK, D), lambda i: (i, 0))],
      out_specs=pl.BlockSpec((8, D), lambda i: (0, 0)),
      out_shape=sds((8, D), f32),
      scratch_shapes=[pltpu.VMEM((8, D), f32)],
  )(alpha)

  m2 = stats2[0] / n_live
  v2 = stats2[1] / n_live - m2 * m2
  iv2 = lax.rsqrt(v2 + EPS)
  st2 = jnp.zeros((8, D), f32).at[0].set(m2).at[1].set(iv2)

  DA = D // 8
  a1, stats3r = pl.pallas_call(
      _tc_c1,
      grid=(EGRID,),
      in_specs=[
          pl.BlockSpec((EBLK, D), lambda i: (i, 0)),
          pl.BlockSpec((8, D), lambda i: (0, 0)),
          pl.BlockSpec((D, DA), lambda i: (0, 0)),
          pl.BlockSpec((1, DA), lambda i: (0, 0)),
      ],
      out_specs=[
          pl.BlockSpec((EBLK, DA), lambda i: (i, 0)),
          pl.BlockSpec((8, DA), lambda i: (0, 0)),
      ],
      out_shape=[sds((EP, DA), f32), sds((8, DA), f32)],
      scratch_shapes=[pltpu.VMEM((8, DA), f32)],
  )(alpha, st2, Wa1.T, ba1[None, :])

  cdead = jnp.maximum((0.0 - m2) * iv2, 0.0) @ Wa1.T + ba1
  n_dead = jnp.float32(EP) - n_live
  s3 = stats3r[0] - n_dead * cdead
  q3 = stats3r[1] - n_dead * cdead * cdead
  m3 = s3 / n_live
  v3 = q3 / n_live - m3 * m3
  iv3 = lax.rsqrt(v3 + EPS)
  st3 = jnp.zeros((8, DA), f32).at[0].set(m3).at[1].set(iv3)

  ae = pl.pallas_call(
      _tc_c2,
      grid=(EGRID,),
      in_specs=[
          pl.BlockSpec((EBLK, DA), lambda i: (i, 0)),
          pl.BlockSpec((8, DA), lambda i: (0, 0)),
          pl.BlockSpec((DA, DA), lambda i: (0, 0)),
          pl.BlockSpec((1, DA), lambda i: (0, 0)),
      ],
      out_specs=pl.BlockSpec((EBLK, DA), lambda i: (i, 0)),
      out_shape=sds((EP, DA), f32),
  )(a1, st3, Wa2.T, ba2[None, :])

  asum_p, cnt_p = pl.kernel(
      _sc_d,
      out_type=[sds((NC, NP, DA), f32), sds((NC, NP), f32)],
      mesh=_mesh,
      compiler_params=pltpu.CompilerParams(needs_layout_passes=False, use_tc_tiling_on_sc=False),
      scratch_types=[
          pltpu.VMEM_SHARED((NP, DA), f32), pltpu.VMEM_SHARED((NP,), f32),
          pltpu.VMEM((CH, DA), f32),
          pltpu.VMEM((CH,), jnp.int32), pltpu.VMEM((CH,), f32),
          pltpu.VMEM((RPT, DA), f32), pltpu.VMEM((RPT,), f32),
      ],
  )(ae, dst, msk)

  asum = asum_p[0] + asum_p[1]
  cnt = cnt_p[0] + cnt_p[1]

  out_p = pl.kernel(
      _sc_e,
      out_type=sds((NC, NP, D), f32),
      mesh=_mesh,
      compiler_params=pltpu.CompilerParams(needs_layout_passes=False, use_tc_tiling_on_sc=False),
      scratch_types=[
          pltpu.VMEM_SHARED((NP, D), f32),
          pltpu.VMEM((CH,), jnp.int32), pltpu.VMEM((CH,), jnp.int32),
          pltpu.VMEM((4, D), f32),
          pltpu.VMEM((CH, DA), f32), pltpu.VMEM((CH, DA), f32),
          pltpu.VMEM((CH, D), f32), pltpu.VMEM((CH, D), f32),
          pltpu.VMEM((CH,), f32), pltpu.VMEM((CH,), f32),
          pltpu.VMEM((CH,), f32),
          pltpu.SemaphoreType.DMA, pltpu.SemaphoreType.DMA,
      ],
  )(hlin_p, ae, asum, r0a, r1a, r2a, src, dst, wp2t)

  out = pl.pallas_call(
      _tc_fin,
      out_shape=sds((N, D), f32),
  )(out_p[0, :N], out_p[1, :N], cnt[:N, None], x, W_out.T)
  return out
